# baseline (speedup 1.0000x reference)
import functools

import jax
import jax.numpy as jnp
from jax.experimental import pallas as pl
from jax.experimental.pallas import tpu as pltpu

EPS = 1e-5
_VMEM_LIMIT = 64 * 1024 * 1024


def _conv3x3_flat(xf, wc_ref, mk_ref, c, hw, w):
    z = jnp.zeros((c, 2 * w), xf.dtype)
    zc = jnp.zeros((c, 1), xf.dtype)
    pf = jnp.concatenate([z, xf, z], axis=1)
    base_m1 = jnp.concatenate([zc, pf[:, :-1]], axis=1) * mk_ref[0:1, :]
    base_p1 = jnp.concatenate([pf[:, 1:], zc], axis=1) * mk_ref[1:2, :]
    b = jnp.concatenate([base_m1, pf, base_p1], axis=0)

    acc = None
    for g in range(3):
        d = jnp.dot(wc_ref[:, 3 * c * g: 3 * c * (g + 1)],
                    b[:, (g + 1) * w: (g + 1) * w + hw],
                    preferred_element_type=jnp.float32)
        acc = d if acc is None else acc + d
    return acc


def _stats2(y):
    s = jnp.sum(y, axis=1, keepdims=True)
    ss = jnp.sum(y * y, axis=1, keepdims=True)
    return jnp.concatenate([s, ss], axis=1)[None]


def _conv1_kernel(x_ref, wc_ref, mk_ref, y_ref, st_ref):
    _, c, h, w = x_ref.shape
    hw = h * w
    xb = x_ref[0].astype(jnp.bfloat16).reshape(c, hw)
    y = _conv3x3_flat(xb, wc_ref, mk_ref, c, hw, w)
    y_ref[...] = y[None].astype(jnp.bfloat16)
    st_ref[...] = _stats2(y)


def _fold_bn(st_ref, g_ref, b_ref, count):
    st = st_ref[...]
    s = jnp.sum(st[:, :, 0], axis=0, keepdims=True)
    ss = jnp.sum(st[:, :, 1], axis=0, keepdims=True)
    mean = s / count
    var = jnp.maximum(ss / count - mean * mean, 0.0)
    scale = g_ref[...] * jax.lax.rsqrt(var + EPS)
    shift = b_ref[...] - mean * scale
    return scale.reshape(-1, 1), shift.reshape(-1, 1)


def _bn_relu_conv2_kernel(w_s, count, y1_ref, wc_ref, mk_ref, st1_ref,
                          g_ref, b_ref, y_ref, st_ref):
    _, c, hw = y1_ref.shape
    sc, sh = _fold_bn(st1_ref, g_ref, b_ref, count)
    r = jnp.maximum(y1_ref[0] * sc.astype(jnp.bfloat16)
                    + sh.astype(jnp.bfloat16), 0.0)
    y = _conv3x3_flat(r, wc_ref, mk_ref, c, hw, w_s)
    y_ref[...] = y[None].astype(jnp.bfloat16)
    st_ref[...] = _stats2(y)


def _bn_add_relu_kernel(count, y2_ref, x_ref, st2_ref, g_ref, b_ref, o_ref):
    _, c, th, w = o_ref.shape
    sc, sh = _fold_bn(st2_ref, g_ref, b_ref, count)
    y2 = y2_ref[0].reshape(c, th, w)[None]
    o_ref[...] = jnp.maximum(y2 * sc.reshape(1, c, 1, 1)
                             + sh.reshape(1, c, 1, 1) + x_ref[...], 0.0)


@jax.jit
def kernel(x_nchw, w1, w2, g1, b1, g2, b2):
    n, c, h, w = x_nchw.shape
    hw = h * w
    count = float(n * hw)
    w1c = jnp.transpose(w1, (2, 0, 1)).reshape(c, 9 * c).astype(jnp.bfloat16)
    w2c = jnp.transpose(w2, (2, 0, 1)).reshape(c, 9 * c).astype(jnp.bfloat16)
    lane = jnp.arange(hw + 4 * w, dtype=jnp.int32) % w
    masks = jnp.stack([(lane != 0), (lane != w - 1)]).astype(jnp.bfloat16)

    img4_spec = pl.BlockSpec((1, c, h, w), lambda i: (i, 0, 0, 0))
    imgf_spec = pl.BlockSpec((1, c, hw), lambda i: (i, 0, 0))
    w_spec = pl.BlockSpec((c, 9 * c), lambda i: (0, 0))
    mk_spec = pl.BlockSpec((2, hw + 4 * w), lambda i: (0, 0))
    gb_spec = pl.BlockSpec((1, c), lambda i: (0, 0))
    stin_spec = pl.BlockSpec((n, c, 2), lambda i: (0, 0, 0))
    st_spec = pl.BlockSpec((1, c, 2), lambda i: (i, 0, 0))
    act_bf16 = jax.ShapeDtypeStruct((n, c, hw), jnp.bfloat16)
    st_shape = jax.ShapeDtypeStruct((n, c, 2), jnp.float32)
    cparams = pltpu.CompilerParams(dimension_semantics=("parallel",),
                                   vmem_limit_bytes=_VMEM_LIMIT)

    y1, st1 = pl.pallas_call(
        _conv1_kernel, grid=(n,),
        in_specs=[img4_spec, w_spec, mk_spec],
        out_specs=(imgf_spec, st_spec),
        out_shape=(act_bf16, st_shape),
        compiler_params=cparams)(x_nchw, w1c, masks)

    y2, st2 = pl.pallas_call(
        functools.partial(_bn_relu_conv2_kernel, w, count),
        grid=(n,),
        in_specs=[imgf_spec, w_spec, mk_spec, stin_spec, gb_spec, gb_spec],
        out_specs=(imgf_spec, st_spec),
        out_shape=(act_bf16, st_shape),
        compiler_params=cparams)(y1, w2c, masks, st1, g1, b1)

    gh3 = 8
    blkf3 = pl.BlockSpec((1, c, hw // gh3), lambda i, j: (i, 0, j))
    blk43 = pl.BlockSpec((1, c, h // gh3, w), lambda i, j: (i, 0, j, 0))
    stin3 = pl.BlockSpec((n, c, 2), lambda i, j: (0, 0, 0))
    gb3 = pl.BlockSpec((1, c), lambda i, j: (0, 0))
    out = pl.pallas_call(
        functools.partial(_bn_add_relu_kernel, count),
        grid=(n, gh3),
        in_specs=[blkf3, blk43, stin3, gb3, gb3],
        out_specs=blk43,
        out_shape=jax.ShapeDtypeStruct((n, c, h, w), jnp.float32),
        compiler_params=pltpu.CompilerParams(
            dimension_semantics=("parallel", "arbitrary"),
            vmem_limit_bytes=_VMEM_LIMIT))(y2, x_nchw, st2, g2, b2)
    return out


# speedup vs baseline: 1.0888x; 1.0888x over previous
import functools

import jax
import jax.numpy as jnp
from jax.experimental import pallas as pl
from jax.experimental.pallas import tpu as pltpu

EPS = 1e-5
_VMEM_LIMIT = 64 * 1024 * 1024


def _conv3x3_flat(xf, wc_ref, mk_ref, c, hw, w):
    z = jnp.zeros((c, 2 * w), xf.dtype)
    zc = jnp.zeros((c, 1), xf.dtype)
    pf = jnp.concatenate([z, xf, z], axis=1)
    base_m1 = jnp.concatenate([zc, pf[:, :-1]], axis=1) * mk_ref[0:1, :]
    base_p1 = jnp.concatenate([pf[:, 1:], zc], axis=1) * mk_ref[1:2, :]
    b = jnp.concatenate([base_m1, pf, base_p1], axis=0)

    acc = None
    for g in range(3):
        d = jnp.dot(wc_ref[:, 3 * c * g: 3 * c * (g + 1)],
                    b[:, (g + 1) * w: (g + 1) * w + hw],
                    preferred_element_type=jnp.float32)
        acc = d if acc is None else acc + d
    return acc


def _stats2(y):
    s = jnp.sum(y, axis=1, keepdims=True)
    ss = jnp.sum(y * y, axis=1, keepdims=True)
    return jnp.concatenate([s, ss], axis=1)[None]


def _conv1_kernel(x_ref, wc_ref, mk_ref, y_ref, st_ref):
    _, c, h, w = x_ref.shape
    hw = h * w
    xb = x_ref[0].astype(jnp.bfloat16).reshape(c, hw)
    y = _conv3x3_flat(xb, wc_ref, mk_ref, c, hw, w)
    y_ref[...] = y[None].astype(jnp.bfloat16)
    st_ref[...] = _stats2(y)


def _bn_relu_conv2_kernel(w_s, y1_ref, wc_ref, mk_ref, sc_ref, sh_ref,
                          y_ref, st_ref):
    _, c, hw = y1_ref.shape
    sc = sc_ref[...].astype(jnp.bfloat16)
    sh = sh_ref[...].astype(jnp.bfloat16)
    r = jnp.maximum(y1_ref[0] * sc + sh, 0.0)
    y = _conv3x3_flat(r, wc_ref, mk_ref, c, hw, w_s)
    y_ref[...] = y[None].astype(jnp.bfloat16)
    st_ref[...] = _stats2(y)


def _bn_add_relu_kernel(y2_ref, x_ref, sc_ref, sh_ref, o_ref):
    _, c, th, w = o_ref.shape
    sc = sc_ref[...].reshape(1, c, 1, 1)
    sh = sh_ref[...].reshape(1, c, 1, 1)
    y2 = y2_ref[0].reshape(c, th, w)[None]
    o_ref[...] = jnp.maximum(y2 * sc + sh + x_ref[...], 0.0)


def _finalize_bn(st, gamma, beta, count):
    s = jnp.sum(st[:, :, 0], axis=0)
    ss = jnp.sum(st[:, :, 1], axis=0)
    mean = s / count
    var = jnp.maximum(ss / count - mean * mean, 0.0)
    scale = gamma.reshape(-1) * jax.lax.rsqrt(var + EPS)
    shift = beta.reshape(-1) - mean * scale
    return scale.reshape(-1, 1), shift.reshape(-1, 1)


@jax.jit
def kernel(x_nchw, w1, w2, g1, b1, g2, b2):
    n, c, h, w = x_nchw.shape
    hw = h * w
    count = float(n * hw)
    w1c = jnp.transpose(w1, (2, 0, 1)).reshape(c, 9 * c).astype(jnp.bfloat16)
    w2c = jnp.transpose(w2, (2, 0, 1)).reshape(c, 9 * c).astype(jnp.bfloat16)
    lane = jnp.arange(hw + 4 * w, dtype=jnp.int32) % w
    masks = jnp.stack([(lane != 0), (lane != w - 1)]).astype(jnp.bfloat16)

    img4_spec = pl.BlockSpec((1, c, h, w), lambda i: (i, 0, 0, 0))
    imgf_spec = pl.BlockSpec((1, c, hw), lambda i: (i, 0, 0))
    w_spec = pl.BlockSpec((c, 9 * c), lambda i: (0, 0))
    mk_spec = pl.BlockSpec((2, hw + 4 * w), lambda i: (0, 0))
    vec_spec = pl.BlockSpec((c, 1), lambda i: (0, 0))
    st_spec = pl.BlockSpec((1, c, 2), lambda i: (i, 0, 0))
    act_bf16 = jax.ShapeDtypeStruct((n, c, hw), jnp.bfloat16)
    st_shape = jax.ShapeDtypeStruct((n, c, 2), jnp.float32)
    cparams = pltpu.CompilerParams(dimension_semantics=("parallel",),
                                   vmem_limit_bytes=_VMEM_LIMIT)

    y1, st1 = pl.pallas_call(
        _conv1_kernel, grid=(n,),
        in_specs=[img4_spec, w_spec, mk_spec],
        out_specs=(imgf_spec, st_spec),
        out_shape=(act_bf16, st_shape),
        compiler_params=cparams)(x_nchw, w1c, masks)
    sc1, sh1 = _finalize_bn(st1, g1, b1, count)

    y2, st2 = pl.pallas_call(
        functools.partial(_bn_relu_conv2_kernel, w),
        grid=(n,),
        in_specs=[imgf_spec, w_spec, mk_spec, vec_spec, vec_spec],
        out_specs=(imgf_spec, st_spec),
        out_shape=(act_bf16, st_shape),
        compiler_params=cparams)(y1, w2c, masks, sc1, sh1)
    sc2, sh2 = _finalize_bn(st2, g2, b2, count)

    gh3 = 4
    blkf3 = pl.BlockSpec((1, c, hw // gh3), lambda i, j: (i, 0, j))
    blk43 = pl.BlockSpec((1, c, h // gh3, w), lambda i, j: (i, 0, j, 0))
    vec3 = pl.BlockSpec((c, 1), lambda i, j: (0, 0))
    out = pl.pallas_call(
        _bn_add_relu_kernel, grid=(n, gh3),
        in_specs=[blkf3, blk43, vec3, vec3],
        out_specs=blk43,
        out_shape=jax.ShapeDtypeStruct((n, c, h, w), jnp.float32),
        compiler_params=pltpu.CompilerParams(
            dimension_semantics=("parallel", "arbitrary"),
            vmem_limit_bytes=_VMEM_LIMIT))(y2, x_nchw, sc2, sh2)
    return out


# speedup vs baseline: 1.1125x; 1.0218x over previous
import functools

import jax
import jax.numpy as jnp
from jax.experimental import pallas as pl
from jax.experimental.pallas import tpu as pltpu

EPS = 1e-5
_VMEM_LIMIT = 64 * 1024 * 1024


def _conv3x3_flat(xf, wc_ref, mk_ref, c, hw, w):
    z = jnp.zeros((c, 2 * w), xf.dtype)
    zc = jnp.zeros((c, 1), xf.dtype)
    pf = jnp.concatenate([z, xf, z], axis=1)
    base_m1 = jnp.concatenate([zc, pf[:, :-1]], axis=1) * mk_ref[0:1, :]
    base_p1 = jnp.concatenate([pf[:, 1:], zc], axis=1) * mk_ref[1:2, :]
    b = jnp.concatenate([base_m1, pf, base_p1], axis=0)

    acc = None
    for g in range(3):
        d = jnp.dot(wc_ref[:, 3 * c * g: 3 * c * (g + 1)],
                    b[:, (g + 1) * w: (g + 1) * w + hw],
                    preferred_element_type=jnp.float32)
        acc = d if acc is None else acc + d
    return acc


def _stats2(y):
    s = jnp.sum(y, axis=1, keepdims=True)
    ss = jnp.sum(y * y, axis=1, keepdims=True)
    return jnp.concatenate([s, ss], axis=1)[None]


def _conv1_kernel(x_ref, wc_ref, mk_ref, y_ref, st_ref):
    _, c, h, w = x_ref.shape
    hw = h * w
    xb = x_ref[0].astype(jnp.bfloat16).reshape(c, hw)
    y = _conv3x3_flat(xb, wc_ref, mk_ref, c, hw, w)
    y_ref[...] = y[None].astype(jnp.bfloat16)
    st_ref[...] = _stats2(y)


def _fold_bn(st_ref, g_ref, b_ref, count):
    st = st_ref[...]
    s = jnp.sum(st[:, :, 0], axis=0, keepdims=True)
    ss = jnp.sum(st[:, :, 1], axis=0, keepdims=True)
    mean = s / count
    var = jnp.maximum(ss / count - mean * mean, 0.0)
    scale = g_ref[...] * jax.lax.rsqrt(var + EPS)
    shift = b_ref[...] - mean * scale
    return scale.reshape(-1, 1), shift.reshape(-1, 1)


def _bn_relu_conv2_kernel(w_s, count, y1_ref, wc_ref, mk_ref, st1_ref,
                          g_ref, b_ref, y_ref, st_ref):
    _, c, hw = y1_ref.shape
    scf, shf = _fold_bn(st1_ref, g_ref, b_ref, count)
    sc = scf.astype(jnp.bfloat16)
    sh = shf.astype(jnp.bfloat16)
    r = jnp.maximum(y1_ref[0] * sc + sh, 0.0)
    y = _conv3x3_flat(r, wc_ref, mk_ref, c, hw, w_s)
    y_ref[...] = y[None].astype(jnp.bfloat16)
    st_ref[...] = _stats2(y)


def _bn_add_relu_kernel(count, y2_ref, x_ref, st2_ref, g_ref, b_ref, o_ref):
    _, c, th, w = o_ref.shape
    scf, shf = _fold_bn(st2_ref, g_ref, b_ref, count)
    sc = scf.reshape(1, c, 1, 1)
    sh = shf.reshape(1, c, 1, 1)
    y2 = y2_ref[0].reshape(c, th, w)[None]
    o_ref[...] = jnp.maximum(y2 * sc + sh + x_ref[...], 0.0)


@jax.jit
def kernel(x_nchw, w1, w2, g1, b1, g2, b2):
    n, c, h, w = x_nchw.shape
    hw = h * w
    count = float(n * hw)
    w1c = jnp.transpose(w1, (2, 0, 1)).reshape(c, 9 * c).astype(jnp.bfloat16)
    w2c = jnp.transpose(w2, (2, 0, 1)).reshape(c, 9 * c).astype(jnp.bfloat16)
    lane = jnp.arange(hw + 4 * w, dtype=jnp.int32) % w
    masks = jnp.stack([(lane != 0), (lane != w - 1)]).astype(jnp.bfloat16)

    img4_spec = pl.BlockSpec((1, c, h, w), lambda i: (i, 0, 0, 0))
    imgf_spec = pl.BlockSpec((1, c, hw), lambda i: (i, 0, 0))
    w_spec = pl.BlockSpec((c, 9 * c), lambda i: (0, 0))
    mk_spec = pl.BlockSpec((2, hw + 4 * w), lambda i: (0, 0))
    gb_spec = pl.BlockSpec((1, c), lambda i: (0, 0))
    stin_spec = pl.BlockSpec((n, c, 2), lambda i: (0, 0, 0))
    st_spec = pl.BlockSpec((1, c, 2), lambda i: (i, 0, 0))
    act_bf16 = jax.ShapeDtypeStruct((n, c, hw), jnp.bfloat16)
    st_shape = jax.ShapeDtypeStruct((n, c, 2), jnp.float32)
    cparams = pltpu.CompilerParams(dimension_semantics=("parallel",),
                                   vmem_limit_bytes=_VMEM_LIMIT)

    y1, st1 = pl.pallas_call(
        _conv1_kernel, grid=(n,),
        in_specs=[img4_spec, w_spec, mk_spec],
        out_specs=(imgf_spec, st_spec),
        out_shape=(act_bf16, st_shape),
        compiler_params=cparams)(x_nchw, w1c, masks)

    y2, st2 = pl.pallas_call(
        functools.partial(_bn_relu_conv2_kernel, w, count),
        grid=(n,),
        in_specs=[imgf_spec, w_spec, mk_spec, stin_spec, gb_spec, gb_spec],
        out_specs=(imgf_spec, st_spec),
        out_shape=(act_bf16, st_shape),
        compiler_params=cparams)(y1, w2c, masks, st1, g1, b1)

    gh3 = 4
    blkf3 = pl.BlockSpec((1, c, hw // gh3), lambda i, j: (i, 0, j))
    blk43 = pl.BlockSpec((1, c, h // gh3, w), lambda i, j: (i, 0, j, 0))
    stin3 = pl.BlockSpec((n, c, 2), lambda i, j: (0, 0, 0))
    gb3 = pl.BlockSpec((1, c), lambda i, j: (0, 0))
    out = pl.pallas_call(
        functools.partial(_bn_add_relu_kernel, count),
        grid=(n, gh3),
        in_specs=[blkf3, blk43, stin3, gb3, gb3],
        out_specs=blk43,
        out_shape=jax.ShapeDtypeStruct((n, c, h, w), jnp.float32),
        compiler_params=pltpu.CompilerParams(
            dimension_semantics=("parallel", "arbitrary"),
            vmem_limit_bytes=_VMEM_LIMIT))(y2, x_nchw, st2, g2, b2)
    return out
